# baseline (speedup 1.0000x reference)
import jax
import jax.numpy as jnp
from jax import lax
from jax.experimental import pallas as pl
from jax.experimental.pallas import tpu as pltpu
from jax.experimental.pallas import tpu_sc as plsc

_EPS = float(jnp.finfo(jnp.float32).eps)
_NEG = float("-inf")
_S, _D, _E, _C = 2048, 4096, 16, 256
_NW = 16
_T = _S // _NW
_Q = 16
_NQ = _T // _Q


def _matmul_kernel(x_ref, w_ref, out_ref):
    out_ref[...] = jax.lax.dot_general(
        x_ref[...], w_ref[...],
        dimension_numbers=(((1,), (1,)), ((), ())),
        preferred_element_type=jnp.float32,
    )


def _sc_gating(logits_hbm, laux_hbm, combine_hbm,
               logits_vf, comb_v, exs_v, stage_v, stage_g, laux_v,
               spm_log, spm_ex, shared_cnt, shared_g,
               logits_s, i12_s, ssum_s,
               cnt1_s, cnt2_s, p1_s, p2_s, tot1_s,
               l1q_s, l2q_s, d2q_s):
    wid = lax.axis_index("s")
    iota = jnp.arange(_E, dtype=jnp.int32)
    zeros = jnp.zeros((_E,), jnp.float32)

    pltpu.sync_copy(logits_hbm.at[pl.ds(wid * (_T * _E), _T * _E)], logits_vf)
    pltpu.sync_copy(logits_vf, spm_log.at[pl.ds(wid * (_T * _E), _T * _E)])
    for e in range(_E):
        cnt1_s[e] = 0
        cnt2_s[e] = 0

    gsum = zeros
    for q in range(_NQ):
        pltpu.sync_copy(
            spm_log.at[pl.ds(wid * (_T * _E) + q * (_Q * _E), _Q * _E)],
            logits_s)

        def a_scal(tq, carry, q=q):
            tb = tq * _E
            best = logits_s[tb]
            bi = jnp.int32(0)
            for e in range(1, _E):
                se = logits_s[tb + e]
                gt = se > best
                best = jnp.where(gt, se, best)
                bi = jnp.where(gt, jnp.int32(e), bi)
            best2 = jnp.float32(_NEG)
            bi2 = jnp.int32(0)
            for e in range(_E):
                se = jnp.where(bi == e, jnp.float32(_NEG), logits_s[tb + e])
                gt = se > best2
                best2 = jnp.where(gt, se, best2)
                bi2 = jnp.where(gt, jnp.int32(e), bi2)
            r1 = cnt1_s[bi]
            cnt1_s[bi] = r1 + 1
            r2 = cnt2_s[bi2]
            cnt2_s[bi2] = r2 + 1
            t = q * _Q + tq
            i12_s[t] = ((bi * _E + bi2) * 128 + r1) * 128 + r2
            return carry

        lax.fori_loop(0, _Q, a_scal, 0)

        def a_vec(tq, carry, q=q):
            t = q * _Q + tq
            i1 = lax.shift_right_logical(i12_s[t], 18)
            m = logits_s[tq * _E + i1]
            row = logits_vf[pl.ds(t * _E, _E)]
            exs_v[pl.ds(tq * _E, _E)] = jnp.exp(row - m)
            return carry

        lax.fori_loop(0, _Q, a_vec, 0)
        pltpu.sync_copy(exs_v, spm_ex.at[pl.ds(wid * (_Q * _E), _Q * _E)])
        pltpu.sync_copy(spm_ex.at[pl.ds(wid * (_Q * _E), _Q * _E)], logits_s)

        def a_ssum(tq, carry, q=q):
            tb = tq * _E
            ssum = logits_s[tb]
            for e in range(1, _E):
                ssum = ssum + logits_s[tb + e]
            ssum_s[tq] = ssum
            return carry

        lax.fori_loop(0, _Q, a_ssum, 0)

        def a_gsum(tq, acc, q=q):
            return acc + exs_v[pl.ds(tq * _E, _E)] / ssum_s[tq]

        gsum = lax.fori_loop(0, _Q, a_gsum, gsum)

    cv1 = zeros
    cv2 = zeros
    for e in range(_E):
        sel = (iota == e)
        cv1 = jnp.where(sel, cnt1_s[e].astype(jnp.float32), cv1)
        cv2 = jnp.where(sel, cnt2_s[e].astype(jnp.float32), cv2)
    stage_v[pl.ds(0, _E)] = cv1
    stage_v[pl.ds(_E, _E)] = cv2
    pltpu.sync_copy(stage_v, shared_cnt.at[pl.ds(wid * (2 * _E), 2 * _E)])
    stage_g[...] = gsum
    pltpu.sync_copy(stage_g, shared_g.at[pl.ds(wid * _E, _E)])
    plsc.subcore_barrier()

    for e in range(_E):
        p1_s[e] = 0
        p2_s[e] = 0
        tot1_s[e] = 0

    def pre(w, carry):
        pltpu.sync_copy(shared_cnt.at[pl.ds(w * (2 * _E), 2 * _E)],
                        logits_s.at[pl.ds(0, 2 * _E)])
        earlier = w < wid
        for e in range(_E):
            c1 = logits_s[e].astype(jnp.int32)
            c2 = logits_s[e + _E].astype(jnp.int32)
            p1_s[e] = p1_s[e] + jnp.where(earlier, c1, 0)
            p2_s[e] = p2_s[e] + jnp.where(earlier, c2, 0)
            tot1_s[e] = tot1_s[e] + c1
        return carry

    lax.fori_loop(0, _NW, pre, 0)

    @pl.when(wid == 0)
    def _laux():
        pltpu.sync_copy(shared_g, logits_s.at[pl.ds(0, _NW * _E)])
        acc = jnp.float32(0.0)
        for e in range(_E):
            mesum = logits_s[e]
            for w in range(1, _NW):
                mesum = mesum + logits_s[w * _E + e]
            acc = acc + (mesum * (1.0 / _S)) * (
                tot1_s[e].astype(jnp.float32) * (1.0 / _S))
        laux_v[...] = jnp.zeros((_E,), jnp.float32) + acc * (1.0 / _E)
        pltpu.sync_copy(laux_v, laux_hbm)

    for q in range(_NQ):
        pltpu.sync_copy(
            spm_log.at[pl.ds(wid * (_T * _E) + q * (_Q * _E), _Q * _E)],
            logits_s)

        def b_scal(tq, carry, q=q):
            t = q * _Q + tq
            pk = i12_s[t]
            i1 = lax.shift_right_logical(pk, 18)
            i2 = jnp.bitwise_and(lax.shift_right_logical(pk, 14), 15)
            r1 = jnp.bitwise_and(lax.shift_right_logical(pk, 7), 127)
            r2 = jnp.bitwise_and(pk, 127)
            loc1 = r1 + p1_s[i1]
            loc2 = r2 + p2_s[i2] + tot1_s[i2]
            l1q_s[tq] = jnp.where(loc1 < _C, loc1, -1)
            l2q_s[tq] = jnp.where(loc2 < _C, loc2, -1)
            d2q_s[tq] = logits_s[tq * _E + i2] - logits_s[tq * _E + i1]
            return carry

        lax.fori_loop(0, _Q, b_scal, 0)

        def b_vec(tq, carry, q=q):
            t = q * _Q + tq
            l1 = jnp.zeros((_E,), jnp.int32) + l1q_s[tq]
            l2 = jnp.zeros((_E,), jnp.int32) + l2q_s[tq]
            exb = jnp.exp(zeros + d2q_s[tq])
            k1 = jnp.where(l1 >= 0, 1.0, 0.0)
            k2 = jnp.where(l2 >= 0, 1.0, 0.0)
            den = jnp.maximum(k1 + k2 * exb, jnp.float32(_EPS))
            g1n = k1 / den
            g2n = k2 * exb / den
            for k in range(_C // _E):
                ci = iota + (k * _E)
                val = (jnp.where(ci == l1, g1n, zeros)
                       + jnp.where(ci == l2, g2n, zeros))
                comb_v[t, pl.ds(k * _E, _E)] = val
            return carry

        lax.fori_loop(0, _Q, b_vec, 0)

    pltpu.sync_copy(comb_v, combine_hbm.at[pl.ds(wid * _T, _T)])


def kernel(input, W):
    S, D = input.shape
    E = W.shape[0]
    C = 2 * S // E
    RB = 512

    logits = pl.pallas_call(
        _matmul_kernel,
        grid=(S // RB,),
        in_specs=[
            pl.BlockSpec((RB, D), lambda i: (i, 0)),
            pl.BlockSpec((E, D), lambda i: (0, 0)),
        ],
        out_specs=pl.BlockSpec((RB, E), lambda i: (i, 0)),
        out_shape=jax.ShapeDtypeStruct((S, E), jnp.float32),
    )(input, W)

    mesh = plsc.VectorSubcoreMesh(
        core_axis_name="c", subcore_axis_name="s", num_cores=1)
    laux, combine = pl.kernel(
        _sc_gating,
        out_type=[
            jax.ShapeDtypeStruct((_E,), jnp.float32),
            jax.ShapeDtypeStruct((_S, _C), jnp.float32),
        ],
        mesh=mesh,
        scratch_types=[
            pltpu.VMEM((_T * _E,), jnp.float32),
            pltpu.VMEM((_T, _C), jnp.float32),
            pltpu.VMEM((_Q * _E,), jnp.float32),
            pltpu.VMEM((2 * _E,), jnp.float32),
            pltpu.VMEM((_E,), jnp.float32),
            pltpu.VMEM((_E,), jnp.float32),
            pltpu.VMEM_SHARED((_NW * _T * _E,), jnp.float32),
            pltpu.VMEM_SHARED((_NW * _Q * _E,), jnp.float32),
            pltpu.VMEM_SHARED((_NW * 2 * _E,), jnp.float32),
            pltpu.VMEM_SHARED((_NW * _E,), jnp.float32),
            pltpu.SMEM((_Q * _E,), jnp.float32),
            pltpu.SMEM((_T,), jnp.int32),
            pltpu.SMEM((_Q,), jnp.float32),
            pltpu.SMEM((_E,), jnp.int32),
            pltpu.SMEM((_E,), jnp.int32),
            pltpu.SMEM((_E,), jnp.int32),
            pltpu.SMEM((_E,), jnp.int32),
            pltpu.SMEM((_E,), jnp.int32),
            pltpu.SMEM((_Q,), jnp.int32),
            pltpu.SMEM((_Q,), jnp.int32),
            pltpu.SMEM((_Q,), jnp.float32),
        ],
    )(logits.reshape(-1))

    combine3 = combine.reshape(S, 1, C)
    return laux[0], combine3, combine3.astype(bool)


# speedup vs baseline: 2.9152x; 2.9152x over previous
import jax
import jax.numpy as jnp
from jax.experimental import pallas as pl
from jax.experimental.pallas import tpu as pltpu


def _matmul_kernel(x_ref, w_ref, out_ref):
    out_ref[...] = jax.lax.dot_general(
        x_ref[...], w_ref[...],
        dimension_numbers=(((1,), (1,)), ((), ())),
        preferred_element_type=jnp.float32,
    )


def _gating_kernel(logits_ref, laux_ref, combine_ref, dispatch_ref):
    S, E = logits_ref.shape
    C = combine_ref.shape[1]
    logits = logits_ref[...]

    row_max = jnp.max(logits, axis=1, keepdims=True)
    unnorm = jnp.exp(logits - row_max)
    gates = unnorm / jnp.sum(unnorm, axis=1, keepdims=True)

    eidx = jax.lax.broadcasted_iota(jnp.int32, (S, E), 1)
    gmax = jnp.max(gates, axis=1, keepdims=True)
    idx1 = jnp.min(jnp.where(gates == gmax, eidx, E), axis=1, keepdims=True)
    mask1 = eidx == idx1
    masked = jnp.where(mask1, -jnp.inf, logits)
    mmax = jnp.max(masked, axis=1, keepdims=True)
    idx2 = jnp.min(jnp.where(masked == mmax, eidx, E), axis=1, keepdims=True)
    mask2 = eidx == idx2
    m1f = mask1.astype(jnp.float32)
    m2f = mask2.astype(jnp.float32)

    B = 256
    ri = jax.lax.broadcasted_iota(jnp.int32, (B, B), 0)
    ci = jax.lax.broadcasted_iota(jnp.int32, (B, B), 1)
    tri = (ri >= ci).astype(jnp.float32)

    def blocked_cumsum(m):
        parts = []
        run = jnp.zeros((1, E), jnp.float32)
        for b in range(S // B):
            blk = m[b * B:(b + 1) * B]
            cs = jax.lax.dot_general(
                tri, blk, dimension_numbers=(((1,), (0,)), ((), ())),
                preferred_element_type=jnp.float32) + run
            parts.append(cs)
            run = run + jnp.sum(blk, axis=0, keepdims=True)
        return jnp.concatenate(parts, axis=0), run

    c1, tot1 = blocked_cumsum(m1f)
    c2, _ = blocked_cumsum(m2f)
    loc1 = c1 - 1.0
    loc2 = c2 - 1.0 + tot1

    me = jnp.sum(gates, axis=0, keepdims=True) / S
    ce = jnp.sum(m1f, axis=0, keepdims=True) / S
    laux_ref[...] = jnp.sum(me * ce, axis=1, keepdims=True) / E

    keep1 = (mask1 & (loc1 < C)).astype(jnp.float32)
    keep2 = (mask2 & (loc2 < C)).astype(jnp.float32)
    g1 = jnp.sum(gates * keep1, axis=1, keepdims=True)
    g2 = jnp.sum(gates * keep2, axis=1, keepdims=True)
    denom = jnp.maximum(g1 + g2, jnp.float32(jnp.finfo(jnp.float32).eps))
    g1n = g1 / denom
    g2n = g2 / denom
    l1 = jnp.sum(loc1 * keep1, axis=1, keepdims=True).astype(jnp.int32)
    l2 = jnp.sum(loc2 * keep2, axis=1, keepdims=True).astype(jnp.int32)
    cap = jax.lax.broadcasted_iota(jnp.int32, (S, C), 1)
    combine = (g1n * (cap == l1).astype(jnp.float32)
               + g2n * (cap == l2).astype(jnp.float32))
    combine_ref[...] = combine
    dispatch_ref[...] = combine != 0.0


def kernel(input, W):
    S, D = input.shape
    E = W.shape[0]
    C = 2 * S // E
    RB = 512

    logits = pl.pallas_call(
        _matmul_kernel,
        grid=(S // RB,),
        in_specs=[
            pl.BlockSpec((RB, D), lambda i: (i, 0)),
            pl.BlockSpec((E, D), lambda i: (0, 0)),
        ],
        out_specs=pl.BlockSpec((RB, E), lambda i: (i, 0)),
        out_shape=jax.ShapeDtypeStruct((S, E), jnp.float32),
    )(input, W)

    laux, combine, dispatch = pl.pallas_call(
        _gating_kernel,
        out_shape=[
            jax.ShapeDtypeStruct((1, 1), jnp.float32),
            jax.ShapeDtypeStruct((S, C), jnp.float32),
            jax.ShapeDtypeStruct((S, C), jnp.bool_),
        ],
    )(logits)

    return laux[0, 0], combine.reshape(S, 1, C), dispatch.reshape(S, 1, C)
